# Initial kernel scaffold; baseline (speedup 1.0000x reference)
import jax
import jax.numpy as jnp
from jax.experimental import pallas as pl


def kernel(x, W):
    raise NotImplementedError("write your pallas kernel here")


# speedup vs baseline: 1.2860x; 1.2860x over previous
import functools

import jax
import jax.numpy as jnp
from jax import lax
from jax.experimental import pallas as pl
from jax.experimental.pallas import tpu as pltpu
from jax.experimental.pallas import tpu_sc as plsc

N_EMB = 8192
DIM = 32
TILE_N = 512
CHUNK_K = 4096


def _dist_argmin_body(x_ref, w_ref, xsq_ref, wsq_ref, idx_ref):
    x = x_ref[...]
    xsq = xsq_ref[...]

    def chunk(k, carry):
        rmin, ridx = carry
        wc = w_ref[pl.ds(k * CHUNK_K, CHUNK_K), :]
        wsq = wsq_ref[:, pl.ds(k * CHUNK_K, CHUNK_K)]
        m = lax.dot_general(x, wc, (((1,), (1,)), ((), ())),
                            preferred_element_type=jnp.float32)
        d = (xsq - 2.0 * m) + wsq
        cmin = jnp.min(d, axis=1, keepdims=True)
        io = lax.broadcasted_iota(jnp.int32, d.shape, 1)
        cidx = jnp.min(jnp.where(d == cmin, io, N_EMB), axis=1,
                       keepdims=True) + k * CHUNK_K
        better = cmin < rmin
        cminq = cmin.astype(jnp.bfloat16).astype(jnp.float32)
        return (jnp.where(better, cminq, rmin),
                jnp.where(better, cidx, ridx))

    init = (jnp.full((TILE_N, 1), jnp.inf, dtype=jnp.float32),
            jnp.zeros((TILE_N, 1), dtype=jnp.int32))
    _, ridx = lax.fori_loop(0, N_EMB // CHUNK_K, chunk, init)
    idx_ref[...] = ridx


def _argmin_indices(xf, W):
    n = xf.shape[0]
    xsq = jnp.sum(xf ** 2, axis=1, keepdims=True)
    wsq = jnp.sum(W ** 2, axis=1)[None, :]
    grid = n // TILE_N
    idx = pl.pallas_call(
        _dist_argmin_body,
        grid=(grid,),
        in_specs=[
            pl.BlockSpec((TILE_N, DIM), lambda i: (i, 0)),
            pl.BlockSpec((N_EMB, DIM), lambda i: (0, 0)),
            pl.BlockSpec((TILE_N, 1), lambda i: (i, 0)),
            pl.BlockSpec((1, N_EMB), lambda i: (0, 0)),
        ],
        out_specs=pl.BlockSpec((TILE_N, 1), lambda i: (i, 0)),
        out_shape=jax.ShapeDtypeStruct((n, 1), jnp.int32),
    )(xf, W, xsq, wsq)
    return idx.reshape(n)


PAD_DIM = 128


def _make_sc_gather(n_tokens):
    info = plsc.get_sparse_core_info()
    nc, ns = info.num_cores, info.num_subcores
    nw = nc * ns
    b_per_w = n_tokens // nw
    mesh = plsc.VectorSubcoreMesh(core_axis_name="c", subcore_axis_name="s")

    @functools.partial(
        pl.kernel, mesh=mesh,
        out_type=jax.ShapeDtypeStruct((n_tokens, PAD_DIM), jnp.float32),
        scratch_types=[
            pltpu.VMEM((b_per_w,), jnp.int32),
            pltpu.VMEM((b_per_w, PAD_DIM), jnp.float32),
            pltpu.SemaphoreType.DMA,
        ],
    )
    def gather(table_hbm, idx_hbm, out_hbm, idx_v, rows_v, sem):
        wid = lax.axis_index("s") * nc + lax.axis_index("c")
        base = wid * b_per_w
        pltpu.sync_copy(idx_hbm.at[pl.ds(base, b_per_w)], idx_v)
        pltpu.async_copy(table_hbm.at[idx_v], rows_v, sem).wait()
        pltpu.sync_copy(rows_v, out_hbm.at[pl.ds(base, b_per_w)])

    return gather


def kernel(x, W):
    b, c, h, w = x.shape
    xf = jnp.transpose(x, (0, 2, 3, 1)).reshape(-1, c)
    idx = _argmin_indices(xf, W)
    Wp = jnp.pad(W, ((0, 0), (0, PAD_DIM - DIM)))
    q = _make_sc_gather(xf.shape[0])(Wp, idx)[:, :DIM]
    return jnp.transpose(q.reshape(b, h, w, c), (0, 3, 1, 2))


# speedup vs baseline: 1.3835x; 1.0758x over previous
import functools

import jax
import jax.numpy as jnp
from jax import lax
from jax.experimental import pallas as pl
from jax.experimental.pallas import tpu as pltpu
from jax.experimental.pallas import tpu_sc as plsc

N_EMB = 8192
DIM = 32
TILE_N = 1024
CHUNK_K = 4096


def _dist_argmin_body(x_ref, w2_ref, xsq_ref, wsq_ref, idx_ref):
    xc = x_ref[0].reshape(DIM, TILE_N)
    x = xc.T
    xsq = xsq_ref[...]
    io = lax.broadcasted_iota(jnp.int32, (1, CHUNK_K), 1).astype(jnp.float32)

    def chunk(k, carry):
        rmin, ridx = carry
        w2c = w2_ref[pl.ds(k * CHUNK_K, CHUNK_K), :]
        wsq = wsq_ref[:, pl.ds(k * CHUNK_K, CHUNK_K)]
        m2 = lax.dot_general(x, w2c, (((1,), (1,)), ((), ())),
                             preferred_element_type=jnp.float32)
        d = (xsq - m2) + wsq
        cmin = jnp.min(d, axis=1, keepdims=True)
        cidxf = jnp.min(jnp.where(d == cmin, io, jnp.inf), axis=1,
                        keepdims=True)
        cidx = cidxf.astype(jnp.int32) + k * CHUNK_K
        better = cmin < rmin
        cminq = cmin.astype(jnp.bfloat16).astype(jnp.float32)
        return (jnp.where(better, cminq, rmin),
                jnp.where(better, cidx, ridx))

    init = (jnp.full((TILE_N, 1), jnp.inf, dtype=jnp.float32),
            jnp.zeros((TILE_N, 1), dtype=jnp.int32))
    _, ridx = lax.fori_loop(0, N_EMB // CHUNK_K, chunk, init)
    idx_ref[...] = ridx


def _argmin_indices(x, W2, xsq, wsq):
    b = x.shape[0]
    n = b * TILE_N
    idx = pl.pallas_call(
        _dist_argmin_body,
        grid=(b,),
        in_specs=[
            pl.BlockSpec((1, DIM, 32, 32), lambda i: (i, 0, 0, 0)),
            pl.BlockSpec((N_EMB, DIM), lambda i: (0, 0)),
            pl.BlockSpec((TILE_N, 1), lambda i: (i, 0)),
            pl.BlockSpec((1, N_EMB), lambda i: (0, 0)),
        ],
        out_specs=pl.BlockSpec((TILE_N, 1), lambda i: (i, 0)),
        out_shape=jax.ShapeDtypeStruct((n, 1), jnp.int32),
    )(x, W2, xsq, wsq)
    return idx.reshape(n)


PAD_DIM = 128


def _make_sc_gather(n_tokens):
    info = plsc.get_sparse_core_info()
    nc, ns = info.num_cores, info.num_subcores
    nw = nc * ns
    b_per_w = n_tokens // nw
    mesh = plsc.VectorSubcoreMesh(core_axis_name="c", subcore_axis_name="s")

    @functools.partial(
        pl.kernel, mesh=mesh,
        out_type=jax.ShapeDtypeStruct((n_tokens, PAD_DIM), jnp.float32),
        scratch_types=[
            pltpu.VMEM((b_per_w,), jnp.int32),
            pltpu.VMEM((b_per_w, PAD_DIM), jnp.float32),
            pltpu.SemaphoreType.DMA,
        ],
    )
    def gather(table_hbm, idx_hbm, out_hbm, idx_v, rows_v, sem):
        wid = lax.axis_index("s") * nc + lax.axis_index("c")
        base = wid * b_per_w
        pltpu.sync_copy(idx_hbm.at[pl.ds(base, b_per_w)], idx_v)
        pltpu.async_copy(table_hbm.at[idx_v], rows_v, sem).wait()
        pltpu.sync_copy(rows_v, out_hbm.at[pl.ds(base, b_per_w)])

    return gather


def _out_layout_body(q_ref, o_ref):
    q = q_ref[:, :DIM]
    o_ref[0] = q.T.reshape(DIM, 32, 32)


def _to_output_layout(qp, b):
    return pl.pallas_call(
        _out_layout_body,
        grid=(b,),
        in_specs=[pl.BlockSpec((TILE_N, PAD_DIM), lambda i: (i, 0))],
        out_specs=pl.BlockSpec((1, DIM, 32, 32), lambda i: (i, 0, 0, 0)),
        out_shape=jax.ShapeDtypeStruct((b, DIM, 32, 32), jnp.float32),
    )(qp)


def kernel(x, W):
    b, c, h, w = x.shape
    xf = jnp.transpose(x, (0, 2, 3, 1)).reshape(-1, c)
    xsq = jnp.sum(xf ** 2, axis=1, keepdims=True)
    wsq = jnp.sum(W ** 2, axis=1)[None, :]
    idx = _argmin_indices(x, 2.0 * W, xsq, wsq)
    Wp = jnp.pad(W, ((0, 0), (0, PAD_DIM - DIM)))
    qp = _make_sc_gather(b * h * w)(Wp, idx)
    return _to_output_layout(qp, b)


# speedup vs baseline: 1.5609x; 1.1282x over previous
import functools

import jax
import jax.numpy as jnp
from jax import lax
from jax.experimental import pallas as pl
from jax.experimental.pallas import tpu as pltpu
from jax.experimental.pallas import tpu_sc as plsc

N_EMB = 8192
DIM = 32
TILE_N = 1024
CHUNK_K = 4096


RG = 64
NS = CHUNK_K // 128


def _dist_argmin_body(x_ref, w2_ref, xsq_ref, wsq_ref, idx_ref):
    xc = x_ref[0].reshape(DIM, TILE_N)
    x = xc.T
    xsq = xsq_ref[...]
    io = lax.broadcasted_iota(jnp.int32, (1, 128), 1).astype(jnp.float32)

    def chunk(k, carry):
        rmin, ridx = carry
        w2c = w2_ref[pl.ds(k * CHUNK_K, CHUNK_K), :]
        wsq = wsq_ref[:, pl.ds(k * CHUNK_K, CHUNK_K)]
        m2 = lax.dot_general(x, w2c, (((1,), (1,)), ((), ())),
                             preferred_element_type=jnp.float32)

        cmin_parts, cidx_parts = [], []
        for rg in range(TILE_N // RG):
            m2rg = m2[rg * RG:(rg + 1) * RG, :]
            xsq_rg = xsq[rg * RG:(rg + 1) * RG, :]

            rv = jnp.full((RG, 128), jnp.inf, dtype=jnp.float32)
            ri = jnp.zeros((RG, 128), dtype=jnp.float32)
            for s in range(NS):
                dstrip = (xsq_rg - m2rg[:, s * 128:(s + 1) * 128]) \
                    + wsq[:, s * 128:(s + 1) * 128]
                iof = io + jnp.float32(s * 128)
                lt = dstrip < rv
                rv = jnp.where(lt, dstrip, rv)
                ri = jnp.where(lt, iof, ri)
            cmin_rg = jnp.min(rv, axis=1, keepdims=True)
            cidx_rg = jnp.min(jnp.where(rv == cmin_rg, ri, jnp.inf), axis=1,
                              keepdims=True)
            cmin_parts.append(cmin_rg)
            cidx_parts.append(cidx_rg)

        cmin = jnp.concatenate(cmin_parts, axis=0)
        cidxf = jnp.concatenate(cidx_parts, axis=0)
        cidx = cidxf.astype(jnp.int32) + k * CHUNK_K
        better = cmin < rmin
        cminq = cmin.astype(jnp.bfloat16).astype(jnp.float32)
        return (jnp.where(better, cminq, rmin),
                jnp.where(better, cidx, ridx))

    init = (jnp.full((TILE_N, 1), jnp.inf, dtype=jnp.float32),
            jnp.zeros((TILE_N, 1), dtype=jnp.int32))
    _, ridx = lax.fori_loop(0, N_EMB // CHUNK_K, chunk, init)
    idx_ref[...] = ridx


def _argmin_indices(x, W2, xsq, wsq):
    b = x.shape[0]
    n = b * TILE_N
    idx = pl.pallas_call(
        _dist_argmin_body,
        grid=(b,),
        in_specs=[
            pl.BlockSpec((1, DIM, 32, 32), lambda i: (i, 0, 0, 0)),
            pl.BlockSpec((N_EMB, DIM), lambda i: (0, 0)),
            pl.BlockSpec((TILE_N, 1), lambda i: (i, 0)),
            pl.BlockSpec((1, N_EMB), lambda i: (0, 0)),
        ],
        out_specs=pl.BlockSpec((TILE_N, 1), lambda i: (i, 0)),
        out_shape=jax.ShapeDtypeStruct((n, 1), jnp.int32),
    )(x, W2, xsq, wsq)
    return idx.reshape(n)


PAD_DIM = 128


def _make_sc_gather(n_tokens):
    info = plsc.get_sparse_core_info()
    nc, ns = info.num_cores, info.num_subcores
    nw = nc * ns
    b_per_w = n_tokens // nw
    mesh = plsc.VectorSubcoreMesh(core_axis_name="c", subcore_axis_name="s")

    @functools.partial(
        pl.kernel, mesh=mesh,
        out_type=jax.ShapeDtypeStruct((n_tokens, PAD_DIM), jnp.float32),
        scratch_types=[
            pltpu.VMEM((b_per_w,), jnp.int32),
            pltpu.VMEM((b_per_w, PAD_DIM), jnp.float32),
            pltpu.SemaphoreType.DMA,
        ],
    )
    def gather(table_hbm, idx_hbm, out_hbm, idx_v, rows_v, sem):
        wid = lax.axis_index("s") * nc + lax.axis_index("c")
        base = wid * b_per_w
        pltpu.sync_copy(idx_hbm.at[pl.ds(base, b_per_w)], idx_v)
        pltpu.async_copy(table_hbm.at[idx_v], rows_v, sem).wait()
        pltpu.sync_copy(rows_v, out_hbm.at[pl.ds(base, b_per_w)])

    return gather


def _out_layout_body(q_ref, o_ref):
    q = q_ref[:, :DIM]
    o_ref[0] = q.T.reshape(DIM, 32, 32)


def _to_output_layout(qp, b):
    return pl.pallas_call(
        _out_layout_body,
        grid=(b,),
        in_specs=[pl.BlockSpec((TILE_N, PAD_DIM), lambda i: (i, 0))],
        out_specs=pl.BlockSpec((1, DIM, 32, 32), lambda i: (i, 0, 0, 0)),
        out_shape=jax.ShapeDtypeStruct((b, DIM, 32, 32), jnp.float32),
    )(qp)


def kernel(x, W):
    b, c, h, w = x.shape
    xf = jnp.transpose(x, (0, 2, 3, 1)).reshape(-1, c)
    xsq = jnp.sum(xf ** 2, axis=1, keepdims=True)
    wsq = jnp.sum(W ** 2, axis=1)[None, :]
    idx = _argmin_indices(x, 2.0 * W, xsq, wsq)
    Wp = jnp.pad(W, ((0, 0), (0, PAD_DIM - DIM)))
    qp = _make_sc_gather(b * h * w)(Wp, idx)
    return _to_output_layout(qp, b)


# speedup vs baseline: 1.5775x; 1.0106x over previous
import functools

import jax
import jax.numpy as jnp
from jax import lax
from jax.experimental import pallas as pl
from jax.experimental.pallas import tpu as pltpu
from jax.experimental.pallas import tpu_sc as plsc

N_EMB = 8192
DIM = 32
TILE_N = 1024
CHUNK_K = 4096


RG = 64
NS = CHUNK_K // 128


def _dist_argmin_body(x_ref, w2_ref, xsq_ref, wsq_ref, idx_ref):
    xc = x_ref[0].reshape(DIM, TILE_N)
    x = xc.T
    xsq = xsq_ref[...]
    io = lax.broadcasted_iota(jnp.int32, (1, 128), 1).astype(jnp.float32)

    def chunk(k, carry):
        rmin, ridx = carry
        w2c = w2_ref[pl.ds(k * CHUNK_K, CHUNK_K), :]
        wsq = wsq_ref[:, pl.ds(k * CHUNK_K, CHUNK_K)]
        m2 = lax.dot_general(x, w2c, (((1,), (1,)), ((), ())),
                             preferred_element_type=jnp.float32)

        cmin_parts, cidx_parts = [], []
        for rg in range(TILE_N // RG):
            m2rg = m2[rg * RG:(rg + 1) * RG, :]
            xsq_rg = xsq[rg * RG:(rg + 1) * RG, :]

            rv = jnp.full((RG, 128), jnp.inf, dtype=jnp.float32)
            ri = jnp.zeros((RG, 128), dtype=jnp.float32)
            for s in range(NS):
                dstrip = (xsq_rg - m2rg[:, s * 128:(s + 1) * 128]) \
                    + wsq[:, s * 128:(s + 1) * 128]
                iof = io + jnp.float32(s * 128)
                lt = dstrip < rv
                rv = jnp.where(lt, dstrip, rv)
                ri = jnp.where(lt, iof, ri)
            cmin_rg = jnp.min(rv, axis=1, keepdims=True)
            cidx_rg = jnp.min(jnp.where(rv == cmin_rg, ri, jnp.inf), axis=1,
                              keepdims=True)
            cmin_parts.append(cmin_rg)
            cidx_parts.append(cidx_rg)

        cmin = jnp.concatenate(cmin_parts, axis=0)
        cidxf = jnp.concatenate(cidx_parts, axis=0)
        cidx = cidxf.astype(jnp.int32) + k * CHUNK_K
        better = cmin < rmin
        cminq = cmin.astype(jnp.bfloat16).astype(jnp.float32)
        return (jnp.where(better, cminq, rmin),
                jnp.where(better, cidx, ridx))

    init = (jnp.full((TILE_N, 1), jnp.inf, dtype=jnp.float32),
            jnp.zeros((TILE_N, 1), dtype=jnp.int32))
    _, ridx = lax.fori_loop(0, N_EMB // CHUNK_K, chunk, init)
    idx_ref[...] = ridx.reshape(TILE_N)


def _argmin_indices(x, W2, xsq, wsq):
    b = x.shape[0]
    n = b * TILE_N
    idx = pl.pallas_call(
        _dist_argmin_body,
        grid=(b,),
        in_specs=[
            pl.BlockSpec((1, DIM, 32, 32), lambda i: (i, 0, 0, 0)),
            pl.BlockSpec((N_EMB, DIM), lambda i: (0, 0)),
            pl.BlockSpec((TILE_N, 1), lambda i: (i, 0)),
            pl.BlockSpec((1, N_EMB), lambda i: (0, 0)),
        ],
        out_specs=pl.BlockSpec((TILE_N,), lambda i: (i,)),
        out_shape=jax.ShapeDtypeStruct((n,), jnp.int32),
    )(x, W2, xsq, wsq)
    return idx


PAD_DIM = 128


def _make_sc_gather(n_tokens):
    info = plsc.get_sparse_core_info()
    nc, ns = info.num_cores, info.num_subcores
    nw = nc * ns
    b_per_w = n_tokens // nw
    mesh = plsc.VectorSubcoreMesh(core_axis_name="c", subcore_axis_name="s")

    @functools.partial(
        pl.kernel, mesh=mesh,
        out_type=jax.ShapeDtypeStruct((n_tokens, PAD_DIM), jnp.float32),
        scratch_types=[
            pltpu.VMEM((b_per_w,), jnp.int32),
            pltpu.VMEM((b_per_w, PAD_DIM), jnp.float32),
            pltpu.SemaphoreType.DMA,
        ],
    )
    def gather(table_hbm, idx_hbm, out_hbm, idx_v, rows_v, sem):
        wid = lax.axis_index("s") * nc + lax.axis_index("c")
        base = wid * b_per_w
        pltpu.sync_copy(idx_hbm.at[pl.ds(base, b_per_w)], idx_v)
        pltpu.async_copy(table_hbm.at[idx_v], rows_v, sem).wait()
        pltpu.sync_copy(rows_v, out_hbm.at[pl.ds(base, b_per_w)])

    return gather


def _out_layout_body(q_ref, o_ref):
    q = q_ref[:, :DIM]
    o_ref[0] = q.T.reshape(DIM, 32, 32)


def _to_output_layout(qp, b):
    return pl.pallas_call(
        _out_layout_body,
        grid=(b,),
        in_specs=[pl.BlockSpec((TILE_N, PAD_DIM), lambda i: (i, 0))],
        out_specs=pl.BlockSpec((1, DIM, 32, 32), lambda i: (i, 0, 0, 0)),
        out_shape=jax.ShapeDtypeStruct((b, DIM, 32, 32), jnp.float32),
    )(qp)


def kernel(x, W):
    b, c, h, w = x.shape
    xf = jnp.transpose(x, (0, 2, 3, 1)).reshape(-1, c)
    xsq = jnp.sum(xf ** 2, axis=1, keepdims=True)
    wsq = jnp.sum(W ** 2, axis=1)[None, :]
    idx = _argmin_indices(x, 2.0 * W, xsq, wsq)
    Wp = jnp.pad(W, ((0, 0), (0, PAD_DIM - DIM)))
    qp = _make_sc_gather(b * h * w)(Wp, idx)
    return _to_output_layout(qp, b)


# speedup vs baseline: 1.6170x; 1.0250x over previous
import functools

import jax
import jax.numpy as jnp
from jax import lax
from jax.experimental import pallas as pl
from jax.experimental.pallas import tpu as pltpu
from jax.experimental.pallas import tpu_sc as plsc

N_EMB = 8192
DIM = 32
TILE_N = 2048
CHUNK_K = 4096


RG = 128
NS = CHUNK_K // 128


def _dist_argmin_body(x_ref, w2_ref, xsq_ref, wsq_ref, idx_ref):
    x = jnp.concatenate(
        [x_ref[0].reshape(DIM, 1024).T, x_ref[1].reshape(DIM, 1024).T],
        axis=0)
    xsq = xsq_ref[...]
    io = lax.broadcasted_iota(jnp.int32, (1, 128), 1).astype(jnp.float32)

    def chunk(k, carry):
        rmin, ridx = carry
        w2c = w2_ref[pl.ds(k * CHUNK_K, CHUNK_K), :]
        wsq = wsq_ref[:, pl.ds(k * CHUNK_K, CHUNK_K)]
        m2 = lax.dot_general(x, w2c, (((1,), (1,)), ((), ())),
                             preferred_element_type=jnp.float32)

        cmin_parts, cidx_parts = [], []
        for rg in range(TILE_N // RG):
            m2rg = m2[rg * RG:(rg + 1) * RG, :]
            xsq_rg = xsq[rg * RG:(rg + 1) * RG, :]

            rv = jnp.full((RG, 128), jnp.inf, dtype=jnp.float32)
            ri = jnp.zeros((RG, 128), dtype=jnp.float32)
            for s in range(NS):
                dstrip = (xsq_rg - m2rg[:, s * 128:(s + 1) * 128]) \
                    + wsq[:, s * 128:(s + 1) * 128]
                iof = io + jnp.float32(s * 128)
                lt = dstrip < rv
                rv = jnp.where(lt, dstrip, rv)
                ri = jnp.where(lt, iof, ri)
            cmin_rg = jnp.min(rv, axis=1, keepdims=True)
            cidx_rg = jnp.min(jnp.where(rv == cmin_rg, ri, jnp.inf), axis=1,
                              keepdims=True)
            cmin_parts.append(cmin_rg)
            cidx_parts.append(cidx_rg)

        cmin = jnp.concatenate(cmin_parts, axis=0)
        cidxf = jnp.concatenate(cidx_parts, axis=0)
        cidx = cidxf.astype(jnp.int32) + k * CHUNK_K
        better = cmin < rmin
        cminq = cmin.astype(jnp.bfloat16).astype(jnp.float32)
        return (jnp.where(better, cminq, rmin),
                jnp.where(better, cidx, ridx))

    init = (jnp.full((TILE_N, 1), jnp.inf, dtype=jnp.float32),
            jnp.zeros((TILE_N, 1), dtype=jnp.int32))
    _, ridx = lax.fori_loop(0, N_EMB // CHUNK_K, chunk, init)
    idx_ref[...] = ridx.reshape(TILE_N)


def _argmin_indices(x, W2, xsq, wsq):
    b = x.shape[0]
    n = b * 1024
    idx = pl.pallas_call(
        _dist_argmin_body,
        grid=(b // 2,),
        in_specs=[
            pl.BlockSpec((2, DIM, 32, 32), lambda i: (i, 0, 0, 0)),
            pl.BlockSpec((N_EMB, DIM), lambda i: (0, 0)),
            pl.BlockSpec((TILE_N, 1), lambda i: (i, 0)),
            pl.BlockSpec((1, N_EMB), lambda i: (0, 0)),
        ],
        out_specs=pl.BlockSpec((TILE_N,), lambda i: (i,)),
        out_shape=jax.ShapeDtypeStruct((n,), jnp.int32),
    )(x, W2, xsq, wsq)
    return idx


PAD_DIM = 128


def _make_sc_gather(n_tokens):
    info = plsc.get_sparse_core_info()
    nc, ns = info.num_cores, info.num_subcores
    nw = nc * ns
    b_per_w = n_tokens // nw
    mesh = plsc.VectorSubcoreMesh(core_axis_name="c", subcore_axis_name="s")

    @functools.partial(
        pl.kernel, mesh=mesh,
        out_type=jax.ShapeDtypeStruct((n_tokens, PAD_DIM), jnp.float32),
        scratch_types=[
            pltpu.VMEM((b_per_w,), jnp.int32),
            pltpu.VMEM((b_per_w, PAD_DIM), jnp.float32),
            pltpu.SemaphoreType.DMA,
        ],
    )
    def gather(table_hbm, idx_hbm, out_hbm, idx_v, rows_v, sem):
        wid = lax.axis_index("s") * nc + lax.axis_index("c")
        base = wid * b_per_w
        pltpu.sync_copy(idx_hbm.at[pl.ds(base, b_per_w)], idx_v)
        pltpu.async_copy(table_hbm.at[idx_v], rows_v, sem).wait()
        pltpu.sync_copy(rows_v, out_hbm.at[pl.ds(base, b_per_w)])

    return gather


def _out_layout_body(q_ref, o_ref):
    q = q_ref[:, :DIM]
    o_ref[0] = q.T.reshape(DIM, 32, 32)


def _to_output_layout(qp, b):
    return pl.pallas_call(
        _out_layout_body,
        grid=(b,),
        in_specs=[pl.BlockSpec((1024, PAD_DIM), lambda i: (i, 0))],
        out_specs=pl.BlockSpec((1, DIM, 32, 32), lambda i: (i, 0, 0, 0)),
        out_shape=jax.ShapeDtypeStruct((b, DIM, 32, 32), jnp.float32),
    )(qp)


def kernel(x, W):
    b, c, h, w = x.shape
    xf = jnp.transpose(x, (0, 2, 3, 1)).reshape(-1, c)
    xsq = jnp.sum(xf ** 2, axis=1, keepdims=True)
    wsq = jnp.sum(W ** 2, axis=1)[None, :]
    idx = _argmin_indices(x, 2.0 * W, xsq, wsq)
    Wp = jnp.pad(W, ((0, 0), (0, PAD_DIM - DIM)))
    qp = _make_sc_gather(b * h * w)(Wp, idx)
    return _to_output_layout(qp, b)
